# Initial kernel scaffold; baseline (speedup 1.0000x reference)
import jax
import jax.numpy as jnp
from jax.experimental import pallas as pl


def kernel(x, ring, params):
    raise NotImplementedError("write your pallas kernel here")


# speedup vs baseline: 1.5668x; 1.5668x over previous
import jax
import jax.numpy as jnp
from jax.experimental import pallas as pl
from jax.experimental.pallas import tpu as pltpu

B = 8
N = 2048
EPS = 1e-5
NEG = -1e30


def _mm(w, x):
    return jnp.dot(w.astype(jnp.bfloat16), x.astype(jnp.bfloat16),
                   preferred_element_type=jnp.float32)


def _stats(sums, sqs, count):
    m = sums / count
    v = sqs / count - m * m
    return m, jax.lax.rsqrt(v + EPS)


def _feat_transform_body(h_ref, tf_ref, pf_ref):
    for b in range(B):
        pf_ref[b] = jax.lax.dot_general(tf_ref[b], h_ref[b],
                                        (((0,), (0,)), ((), ())),
                                        precision=jax.lax.Precision.HIGHEST,
                                        preferred_element_type=jnp.float32)


def _tail_body(pf_ref, ids_ref, g1w, g1b, g2w, g2b, wr, rb, g_ref, rf_ref):
    cnt = float(B * N)
    rs = jnp.zeros((128, 1), jnp.float32)
    rq = jnp.zeros((128, 1), jnp.float32)
    smax = []
    for b in range(B):
        rv = _mm(wr[...], pf_ref[b]) + rb[...]
        rs = rs + rv.sum(axis=1, keepdims=True)
        rq = rq + (rv * rv).sum(axis=1, keepdims=True)
        ids2 = ids_ref[b:b + 1]
        cols = [jnp.where(ids2 == rr, rv, NEG).max(axis=1, keepdims=True)
                for rr in range(16)]
        smax.append(jnp.concatenate(cols, axis=1))
    mr, rr_ = _stats(rs, rq, cnt)
    for b in range(B):
        ids2 = ids_ref[b:b + 1]
        onehot = (jax.lax.broadcasted_iota(jnp.int32, (16, N), 0)
                  == ids2).astype(jnp.float32)
        rf_ref[b] = _mm((smax[b] - mr) * rr_, onehot)
    y1 = [_mm(g1w[...], pf_ref[b]) + g1b[...] for b in range(B)]
    s = sum(y.sum(axis=1, keepdims=True) for y in y1)
    q = sum((y * y).sum(axis=1, keepdims=True) for y in y1)
    m, r = _stats(s, q, cnt)
    s2 = jnp.zeros((1024, 1), jnp.float32)
    q2 = jnp.zeros((1024, 1), jnp.float32)
    mx = []
    for b in range(B):
        n = jax.nn.relu((y1[b] - m) * r)
        y = _mm(g2w[...], n) + g2b[...]
        s2 = s2 + y.sum(axis=1, keepdims=True)
        q2 = q2 + (y * y).sum(axis=1, keepdims=True)
        mx.append(y.max(axis=1, keepdims=True))
    m2, r2 = _stats(s2, q2, cnt)
    for b in range(B):
        g_ref[b] = (mx[b] - m2) * r2


def _assemble_body(pf_ref, rf_ref, g_ref, out_ref):
    out_ref[0, :64] = pf_ref[0]
    out_ref[0, 64:192] = rf_ref[0]
    out_ref[0, 192:] = jnp.broadcast_to(g_ref[0], (1024, out_ref.shape[2]))


def _call(body, out_shape, *args):
    return pl.pallas_call(
        body,
        out_shape=out_shape,
        compiler_params=pltpu.CompilerParams(
            vmem_limit_bytes=120 * 1024 * 1024),
    )(*args)


def _col(v):
    return v.reshape(-1, 1)


def _conv_bn_relu(x, W, b):
    y = jnp.einsum('oc,bcn->bon', W, x) + b[None, :, None]
    m = jnp.mean(y, axis=(0, 2), keepdims=True)
    v = jnp.var(y, axis=(0, 2), keepdims=True)
    return jax.nn.relu((y - m) / jnp.sqrt(v + EPS))


def _bnfc(x):
    m = jnp.mean(x, axis=0, keepdims=True)
    v = jnp.var(x, axis=0, keepdims=True)
    return (x - m) / jnp.sqrt(v + EPS)


def _stn_tower(x, p, pre, k):
    h = _conv_bn_relu(x, p[pre + 'c1w'], p[pre + 'c1b'])
    h = _conv_bn_relu(h, p[pre + 'c2w'], p[pre + 'c2b'])
    h = _conv_bn_relu(h, p[pre + 'c3w'], p[pre + 'c3b'])
    h = jnp.max(h, axis=2)
    h = jax.nn.relu(_bnfc(h @ p[pre + 'f1w'].T + p[pre + 'f1b']))
    h = jax.nn.relu(_bnfc(h @ p[pre + 'f2w'].T + p[pre + 'f2b']))
    h = h @ p[pre + 'f3w'].T + p[pre + 'f3b']
    h = h + jnp.eye(k, dtype=h.dtype).reshape(-1)[None, :]
    return h


def kernel(x, ring, params):
    p = params
    x = x.astype(jnp.float32)
    ids = ring.astype(jnp.int32)

    t9 = _stn_tower(x, p, 's_', 3)
    trans = t9.reshape(B, 3, 3)
    xt = jnp.swapaxes(x, 1, 2)
    xyz = jnp.matmul(xt[:, :, :3], trans)
    xt = jnp.concatenate([xyz, xt[:, :, 3:]], axis=2)
    x2 = jnp.swapaxes(xt, 1, 2)
    h = _conv_bn_relu(x2, p['c1w'], p['c1b'])

    t4096 = _stn_tower(h, p, 'f_', 64)
    tf = t4096.reshape(B, 64, 64)

    pf = _call(
        _feat_transform_body, jax.ShapeDtypeStruct((B, 64, N), jnp.float32),
        h, tf)

    rwb = p['rw'].reshape(16, 8, 4)
    wr = jax.scipy.linalg.block_diag(*[rwb[g] for g in range(16)])

    g, rf = _call(
        _tail_body,
        (jax.ShapeDtypeStruct((B, 1024, 1), jnp.float32),
         jax.ShapeDtypeStruct((B, 128, N), jnp.float32)),
        pf, ids, p['g1w'], _col(p['g1b']), p['g2w'], _col(p['g2b']),
        wr, _col(p['rb']))

    nt = 4
    out = pl.pallas_call(
        _assemble_body,
        grid=(B, nt),
        in_specs=[
            pl.BlockSpec((1, 64, N // nt), lambda b, t: (b, 0, t)),
            pl.BlockSpec((1, 128, N // nt), lambda b, t: (b, 0, t)),
            pl.BlockSpec((1, 1024, 1), lambda b, t: (b, 0, 0)),
        ],
        out_specs=pl.BlockSpec((1, 1216, N // nt), lambda b, t: (b, 0, t)),
        out_shape=jax.ShapeDtypeStruct((B, 1216, N), jnp.float32),
        compiler_params=pltpu.CompilerParams(
            vmem_limit_bytes=120 * 1024 * 1024),
    )(pf, rf, g)
    return out
